# Initial kernel scaffold; baseline (speedup 1.0000x reference)
import jax
import jax.numpy as jnp
from jax.experimental import pallas as pl


def kernel(x, r):
    raise NotImplementedError("write your pallas kernel here")


# speedup vs baseline: 1.0459x; 1.0459x over previous
import jax
import jax.numpy as jnp
from jax import lax
from jax.experimental import pallas as pl
from jax.experimental.pallas import tpu as pltpu
from jax.experimental.pallas import tpu_sc as plsc

B, H, W, C = 8, 56, 56, 192
HW = H * W
ROWS = B * HW
NW = 32
RPW = ROWS // NW
WPB = HW // RPW
CH = 112
NCH = RPW // CH
LANES = 16


def _body(xf_hbm, r_hbm, out_hbm, idx_v, buf0, buf1, gsem0, gsem1, wsem0, wsem1):
    wid = lax.axis_index("s") * 2 + lax.axis_index("c")
    b = wid // WPB
    p0 = (wid % WPB) * RPW

    pltpu.sync_copy(r_hbm.at[pl.ds(p0, RPW)], idx_v)
    off = b * HW
    for i in range(RPW // LANES):
        sl = pl.ds(i * LANES, LANES)
        idx_v[sl] = idx_v[sl] + off

    base = wid * RPW
    bufs = (buf0, buf1)
    gsems = (gsem0, gsem1)
    wsems = (wsem0, wsem1)

    def start_gather(c):
        return pltpu.async_copy(
            xf_hbm.at[idx_v.at[pl.ds(c * CH, CH)]], bufs[c % 2], gsems[c % 2]
        )

    def start_write(c):
        return pltpu.async_copy(
            bufs[c % 2], out_hbm.at[pl.ds(base + c * CH, CH)], wsems[c % 2]
        )

    gathers = [None, None]
    writes = [None, None]
    gathers[0] = start_gather(0)
    for c in range(NCH):
        n = c + 1
        if n < NCH:
            if writes[n % 2] is not None:
                writes[n % 2].wait()
            gathers[n % 2] = start_gather(n)
        gathers[c % 2].wait()
        writes[c % 2] = start_write(c)
    writes[(NCH - 1) % 2].wait()
    writes[(NCH - 2) % 2].wait()


@jax.jit
def kernel(x, r):
    xf = x.reshape(ROWS, C)
    r = r.astype(jnp.int32)
    mesh = plsc.VectorSubcoreMesh(core_axis_name="c", subcore_axis_name="s")
    out = pl.kernel(
        _body,
        out_type=jax.ShapeDtypeStruct((ROWS, C), jnp.float32),
        mesh=mesh,
        compiler_params=pltpu.CompilerParams(use_tc_tiling_on_sc=False),
        scratch_types=[
            pltpu.VMEM((RPW,), jnp.int32),
            pltpu.VMEM((CH, C), jnp.float32),
            pltpu.VMEM((CH, C), jnp.float32),
            pltpu.SemaphoreType.DMA,
            pltpu.SemaphoreType.DMA,
            pltpu.SemaphoreType.DMA,
            pltpu.SemaphoreType.DMA,
        ],
    )(xf, r)
    return out.reshape(B, H, W, C)


# speedup vs baseline: 1.0528x; 1.0066x over previous
import jax
import jax.numpy as jnp
from jax import lax
from jax.experimental import pallas as pl
from jax.experimental.pallas import tpu as pltpu
from jax.experimental.pallas import tpu_sc as plsc

B, H, W, C = 8, 56, 56, 192
HW = H * W
ROWS = B * HW
NW = 32
RPW = ROWS // NW
WPB = HW // RPW
CH = 112
NCH = RPW // CH
NBUF = 4
LANES = 16


def _body(xf_hbm, r_hbm, out_hbm, idx_v, bufs, gsems, wsems):
    wid = lax.axis_index("s") * 2 + lax.axis_index("c")
    b = wid // WPB
    p0 = (wid % WPB) * RPW

    pltpu.sync_copy(r_hbm.at[pl.ds(p0, RPW)], idx_v)
    off = b * HW
    for i in range(RPW // LANES):
        sl = pl.ds(i * LANES, LANES)
        idx_v[sl] = idx_v[sl] + off

    base = wid * RPW

    def start_gather(c):
        return pltpu.async_copy(
            xf_hbm.at[idx_v.at[pl.ds(c * CH, CH)]], bufs[c % NBUF], gsems[c % NBUF]
        )

    def start_write(c):
        return pltpu.async_copy(
            bufs[c % NBUF], out_hbm.at[pl.ds(base + c * CH, CH)], wsems[c % NBUF]
        )

    gathers = [None] * NBUF
    writes = [None] * NBUF
    for j in range(min(NBUF, NCH)):
        gathers[j] = start_gather(j)
    for c in range(NCH):
        gathers[c % NBUF].wait()
        writes[c % NBUF] = start_write(c)
        n = c + NBUF
        if n < NCH:
            writes[n % NBUF].wait()
            gathers[n % NBUF] = start_gather(n)
    for j in range(max(0, NCH - NBUF), NCH):
        writes[j % NBUF].wait()


@jax.jit
def kernel(x, r):
    xf = x.reshape(ROWS, C)
    r = r.astype(jnp.int32)
    mesh = plsc.VectorSubcoreMesh(core_axis_name="c", subcore_axis_name="s")
    out = pl.kernel(
        _body,
        out_type=jax.ShapeDtypeStruct((ROWS, C), jnp.float32),
        mesh=mesh,
        compiler_params=pltpu.CompilerParams(use_tc_tiling_on_sc=False),
        scratch_types=[
            pltpu.VMEM((RPW,), jnp.int32),
            [pltpu.VMEM((CH, C), jnp.float32) for _ in range(NBUF)],
            [pltpu.SemaphoreType.DMA for _ in range(NBUF)],
            [pltpu.SemaphoreType.DMA for _ in range(NBUF)],
        ],
    )(xf, r)
    return out.reshape(B, H, W, C)
